# baseline (speedup 1.0000x reference)
import functools

import jax
import jax.numpy as jnp
from jax.experimental import pallas as pl
from jax.experimental.pallas import tpu as pltpu

_EPS = 1e-5


def _p1_kernel(x_ref, mb_ref, h_ref, st_ref, *, T, V, C):
    xb = x_ref[0]
    hm = jnp.dot(xb.astype(jnp.bfloat16), mb_ref[...],
                 preferred_element_type=jnp.float32)
    h_ref[0] = hm.astype(jnp.bfloat16)
    s = jnp.sum(hm, axis=0, keepdims=True)
    q = jnp.sum(hm * hm, axis=0, keepdims=True)
    st_ref[0] = jnp.concatenate(
        [s, q, jnp.zeros((6, V * C), jnp.float32)], axis=0)


def _temporal(hb, aff_ref, wt_ref, at_ref, V, C):
    g = jnp.maximum(hb.astype(jnp.float32) * aff_ref[0:1, :]
                    + aff_ref[1:2, :], 0.0)
    zs = []
    for v in range(V):
        zs.append(jnp.dot(g[:, v * C:(v + 1) * C].astype(jnp.bfloat16),
                          wt_ref[...],
                          preferred_element_type=jnp.float32))
    z = jnp.concatenate(zs, axis=1).astype(jnp.bfloat16)
    return jnp.dot(at_ref[...], z,
                   preferred_element_type=jnp.float32)


def _p2_kernel(h_ref, aff_ref, wt_ref, at_ref, st_ref, *, T, V, C):
    h2 = _temporal(h_ref[0], aff_ref, wt_ref, at_ref, V, C)
    s = jnp.sum(h2, axis=0, keepdims=True)
    q = jnp.sum(h2 * h2, axis=0, keepdims=True)
    st_ref[0] = jnp.concatenate(
        [s, q, jnp.zeros((6, V * C), jnp.float32)], axis=0)


def _p3_kernel(h_ref, aff_ref, wt_ref, at_ref, aff2_ref, out_ref, *, T, V, C):
    h2 = _temporal(h_ref[0], aff_ref, wt_ref, at_ref, V, C)
    out_ref[0] = jnp.maximum(h2 * aff2_ref[0:1, :] + aff2_ref[1:2, :], 0.0)


def _bn_affine(stb, n, gamma, beta, V, C):
    st = stb.sum(axis=0)
    s = st[0].reshape(V, C).sum(axis=0)
    q = st[1].reshape(V, C).sum(axis=0)
    mean = s / n
    var = q / n - mean * mean
    inv = gamma * jax.lax.rsqrt(var + _EPS)
    aff = jnp.stack([inv, beta - mean * inv])
    return jnp.tile(aff, (1, V))


def kernel(x, adj, edge_importance, W1, b1, W2, b2, W3, b3, Wt, bt, gamma, beta):
    B, T, V, C = x.shape
    f32 = jnp.float32
    bf16 = jnp.bfloat16
    VC = V * C
    n = float(B * T * V)
    xf = x.reshape(B, T, VC)

    A = adj * edge_importance
    Mb = (jnp.kron(A[0].T, W1) + jnp.kron(A[1].T, W2)
          + jnp.kron(A[2].T, W3)).astype(bf16)
    r = jnp.arange(T)
    At = (jnp.abs(r[:, None] - r[None, :]) <= 4).astype(bf16)

    params = pltpu.CompilerParams(dimension_semantics=("parallel",))
    small = lambda shp: pl.BlockSpec(shp, lambda b: (0,) * len(shp))
    blk = pl.BlockSpec((1, T, VC), lambda b: (b, 0, 0))

    p1 = pl.pallas_call(
        functools.partial(_p1_kernel, T=T, V=V, C=C),
        grid=(B,),
        in_specs=[blk, small((VC, VC))],
        out_specs=[blk, pl.BlockSpec((1, 8, VC), lambda b: (b, 0, 0))],
        out_shape=[jax.ShapeDtypeStruct((B, T, VC), bf16),
                   jax.ShapeDtypeStruct((B, 8, VC), f32)],
        compiler_params=params,
    )
    h, st1 = p1(xf, Mb)
    aff1 = _bn_affine(st1, n, gamma, beta, V, C)

    p2 = pl.pallas_call(
        functools.partial(_p2_kernel, T=T, V=V, C=C),
        grid=(B,),
        in_specs=[blk, small((2, VC)), small((C, C)), small((T, T))],
        out_specs=pl.BlockSpec((1, 8, VC), lambda b: (b, 0, 0)),
        out_shape=jax.ShapeDtypeStruct((B, 8, VC), f32),
        compiler_params=params,
    )
    st2 = p2(h, aff1, Wt.astype(bf16), At)
    aff2 = _bn_affine(st2, n, gamma, beta, V, C)

    p3 = pl.pallas_call(
        functools.partial(_p3_kernel, T=T, V=V, C=C),
        grid=(B,),
        in_specs=[blk, small((2, VC)), small((C, C)), small((T, T)),
                  small((2, VC))],
        out_specs=blk,
        out_shape=jax.ShapeDtypeStruct((B, T, VC), f32),
        compiler_params=params,
    )
    out = p3(h, aff1, Wt.astype(bf16), At, aff2)
    return out.reshape(B, T, V, C)


# speedup vs baseline: 1.0363x; 1.0363x over previous
import functools

import jax
import jax.numpy as jnp
from jax.experimental import pallas as pl
from jax.experimental.pallas import tpu as pltpu

_EPS = 1e-5


def _p1_kernel(x_ref, mb_ref, h_ref, st_ref, *, T, V, C):
    xb = x_ref[0]
    hm = jnp.dot(xb.astype(jnp.bfloat16), mb_ref[...],
                 preferred_element_type=jnp.float32)
    h_ref[0] = hm.astype(jnp.bfloat16)
    s = jnp.sum(hm, axis=0, keepdims=True)
    q = jnp.sum(hm * hm, axis=0, keepdims=True)
    part = jnp.concatenate(
        [s, q, jnp.zeros((6, V * C), jnp.float32)], axis=0)

    @pl.when(pl.program_id(0) == 0)
    def _():
        st_ref[...] = jnp.zeros_like(st_ref)

    st_ref[...] += part


def _p2_kernel(h_ref, aff_ref, wt_ref, at_ref, h2_ref, st_ref, *, T, V, C):
    hb = h_ref[0]
    g = jnp.maximum(hb.astype(jnp.float32) * aff_ref[0:1, :]
                    + aff_ref[1:2, :], 0.0)
    zs = []
    for v in range(V):
        zs.append(jnp.dot(g[:, v * C:(v + 1) * C].astype(jnp.bfloat16),
                          wt_ref[...],
                          preferred_element_type=jnp.float32))
    z = jnp.concatenate(zs, axis=1).astype(jnp.bfloat16)
    h2 = jnp.dot(at_ref[...], z,
                 preferred_element_type=jnp.float32)
    h2_ref[0] = h2.astype(jnp.bfloat16)
    s = jnp.sum(h2, axis=0, keepdims=True)
    q = jnp.sum(h2 * h2, axis=0, keepdims=True)
    part = jnp.concatenate(
        [s, q, jnp.zeros((6, V * C), jnp.float32)], axis=0)

    @pl.when(pl.program_id(0) == 0)
    def _():
        st_ref[...] = jnp.zeros_like(st_ref)

    st_ref[...] += part


def _p3_kernel(h2_ref, aff_ref, out_ref, *, T, V, C):
    hb = h2_ref[0]
    out_ref[0] = jnp.maximum(hb.astype(jnp.float32) * aff_ref[0:1, :]
                             + aff_ref[1:2, :], 0.0)


def _bn_affine(st, n, gamma, beta, V, C):
    s = st[0].reshape(V, C).sum(axis=0)
    q = st[1].reshape(V, C).sum(axis=0)
    mean = s / n
    var = q / n - mean * mean
    inv = gamma * jax.lax.rsqrt(var + _EPS)
    aff = jnp.stack([inv, beta - mean * inv])
    return jnp.tile(aff, (1, V))


def kernel(x, adj, edge_importance, W1, b1, W2, b2, W3, b3, Wt, bt, gamma, beta):
    B, T, V, C = x.shape
    f32 = jnp.float32
    bf16 = jnp.bfloat16
    VC = V * C
    n = float(B * T * V)
    xf = x.reshape(B, T, VC)

    A = adj * edge_importance
    Mb = (jnp.kron(A[0].T, W1) + jnp.kron(A[1].T, W2)
          + jnp.kron(A[2].T, W3)).astype(bf16)
    r = jnp.arange(T)
    At = (jnp.abs(r[:, None] - r[None, :]) <= 4).astype(bf16)

    params = pltpu.CompilerParams(dimension_semantics=("arbitrary",))
    small = lambda shp: pl.BlockSpec(shp, lambda b: (0,) * len(shp))
    blk = pl.BlockSpec((1, T, VC), lambda b: (b, 0, 0))

    p1 = pl.pallas_call(
        functools.partial(_p1_kernel, T=T, V=V, C=C),
        grid=(B,),
        in_specs=[blk, small((VC, VC))],
        out_specs=[blk, small((8, VC))],
        out_shape=[jax.ShapeDtypeStruct((B, T, VC), bf16),
                   jax.ShapeDtypeStruct((8, VC), f32)],
        compiler_params=params,
    )
    h, st1 = p1(xf, Mb)
    aff1 = _bn_affine(st1, n, gamma, beta, V, C)

    p2 = pl.pallas_call(
        functools.partial(_p2_kernel, T=T, V=V, C=C),
        grid=(B,),
        in_specs=[blk, small((2, VC)), small((C, C)), small((T, T))],
        out_specs=[blk, small((8, VC))],
        out_shape=[jax.ShapeDtypeStruct((B, T, VC), bf16),
                   jax.ShapeDtypeStruct((8, VC), f32)],
        compiler_params=params,
    )
    h2, st2 = p2(h, aff1, Wt.astype(bf16), At)
    aff2 = _bn_affine(st2, n, gamma, beta, V, C)

    p3 = pl.pallas_call(
        functools.partial(_p3_kernel, T=T, V=V, C=C),
        grid=(B,),
        in_specs=[blk, small((2, VC))],
        out_specs=blk,
        out_shape=jax.ShapeDtypeStruct((B, T, VC), f32),
        compiler_params=params,
    )
    out = p3(h2, aff2)
    return out.reshape(B, T, V, C)
